# Initial kernel scaffold; baseline (speedup 1.0000x reference)
import jax
import jax.numpy as jnp
from jax.experimental import pallas as pl


def kernel(token_ids, weight):
    raise NotImplementedError("write your pallas kernel here")


# speedup vs baseline: 1.8480x; 1.8480x over previous
import functools

import jax
import jax.numpy as jnp
from jax import lax
from jax.experimental import pallas as pl
from jax.experimental.pallas import tpu as pltpu
from jax.experimental.pallas import tpu_sc as plsc

EMB_D = 64
CHUNK = 128
NB = 4


@functools.lru_cache(maxsize=None)
def _build(n_tokens: int, vocab: int, d: int):
    info = plsc.get_sparse_core_info()
    nc, ns = info.num_cores, info.num_subcores
    nw = nc * ns
    assert n_tokens % (nw * CHUNK * NB) == 0
    chunks_per_w = n_tokens // (nw * CHUNK)
    rows_per_w = chunks_per_w * CHUNK

    mesh = plsc.VectorSubcoreMesh(
        core_axis_name="c", subcore_axis_name="s",
        num_cores=nc, num_subcores=ns,
    )

    @functools.partial(
        pl.kernel,
        out_type=jax.ShapeDtypeStruct((n_tokens, d), jnp.float32),
        mesh=mesh,
        scratch_types=[
            pltpu.VMEM((chunks_per_w, CHUNK), jnp.int32),
            pltpu.VMEM((NB, CHUNK, d), jnp.float32),
        ] + [pltpu.SemaphoreType.DMA] * (2 * NB),
        compiler_params=pltpu.CompilerParams(use_tc_tiling_on_sc=False),
    )
    def k(idx_hbm, table_hbm, out_hbm, idx_v, rows_v, *sems):
        gsem, ssem = sems[:NB], sems[NB:]
        wid = lax.axis_index("s") * nc + lax.axis_index("c")
        pltpu.sync_copy(idx_hbm.at[pl.ds(wid * chunks_per_w, chunks_per_w)],
                        idx_v)
        out_row0 = wid * rows_per_w

        def body(i, _):
            g0 = i * NB
            gathers = []
            for b in range(NB):
                cp = pltpu.async_copy(
                    table_hbm.at[idx_v.at[g0 + b]], rows_v.at[b], gsem[b])
                gathers.append(cp)
            stores = []
            for b in range(NB):
                gathers[b].wait()
                sp = pltpu.async_copy(
                    rows_v.at[b],
                    out_hbm.at[pl.ds(out_row0 + (g0 + b) * CHUNK, CHUNK)],
                    ssem[b])
                stores.append(sp)
            for b in range(NB):
                stores[b].wait()
            return 0

        lax.fori_loop(0, chunks_per_w // NB, body, 0)

    return k


def kernel(token_ids, weight):
    n_tokens = token_ids.shape[0] * token_ids.shape[1]
    idx2d = token_ids.reshape(n_tokens // CHUNK, CHUNK).astype(jnp.int32)
    k = _build(n_tokens, weight.shape[0], weight.shape[1])
    out = k(idx2d, weight)
    return out.reshape(*token_ids.shape, weight.shape[1])


# speedup vs baseline: 1.8869x; 1.0211x over previous
import functools

import jax
import jax.numpy as jnp
from jax import lax
from jax.experimental import pallas as pl
from jax.experimental.pallas import tpu as pltpu
from jax.experimental.pallas import tpu_sc as plsc

EMB_D = 64
CHUNK = 128
NB = 8


@functools.lru_cache(maxsize=None)
def _build(n_tokens: int, vocab: int, d: int):
    info = plsc.get_sparse_core_info()
    nc, ns = info.num_cores, info.num_subcores
    nw = nc * ns
    assert n_tokens % (nw * CHUNK * NB) == 0
    chunks_per_w = n_tokens // (nw * CHUNK)
    rows_per_w = chunks_per_w * CHUNK

    mesh = plsc.VectorSubcoreMesh(
        core_axis_name="c", subcore_axis_name="s",
        num_cores=nc, num_subcores=ns,
    )

    @functools.partial(
        pl.kernel,
        out_type=jax.ShapeDtypeStruct((n_tokens, d), jnp.float32),
        mesh=mesh,
        scratch_types=[
            pltpu.VMEM((chunks_per_w, CHUNK), jnp.int32),
            pltpu.VMEM((NB, CHUNK, d), jnp.float32),
        ] + [pltpu.SemaphoreType.DMA] * (2 * NB),
        compiler_params=pltpu.CompilerParams(use_tc_tiling_on_sc=False),
    )
    def k(idx_hbm, table_hbm, out_hbm, idx_v, rows_v, *sems):
        gsem, ssem = sems[:NB], sems[NB:]
        wid = lax.axis_index("s") * nc + lax.axis_index("c")
        pltpu.sync_copy(idx_hbm.at[pl.ds(wid * chunks_per_w, chunks_per_w)],
                        idx_v)
        out_row0 = wid * rows_per_w

        def body(i, _):
            g0 = i * NB
            gathers = []
            for b in range(NB):
                cp = pltpu.async_copy(
                    table_hbm.at[idx_v.at[g0 + b]], rows_v.at[b], gsem[b])
                gathers.append(cp)
            stores = []
            for b in range(NB):
                gathers[b].wait()
                sp = pltpu.async_copy(
                    rows_v.at[b],
                    out_hbm.at[pl.ds(out_row0 + (g0 + b) * CHUNK, CHUNK)],
                    ssem[b])
                stores.append(sp)
            for b in range(NB):
                stores[b].wait()
            return 0

        lax.fori_loop(0, chunks_per_w // NB, body, 0)

    return k


def kernel(token_ids, weight):
    n_tokens = token_ids.shape[0] * token_ids.shape[1]
    idx2d = token_ids.reshape(n_tokens // CHUNK, CHUNK).astype(jnp.int32)
    k = _build(n_tokens, weight.shape[0], weight.shape[1])
    out = k(idx2d, weight)
    return out.reshape(*token_ids.shape, weight.shape[1])
